# baseline (speedup 1.0000x reference)
import functools

import jax
import jax.numpy as jnp
from jax import lax
from jax.experimental import pallas as pl
from jax.experimental.pallas import tpu as pltpu
from jax.experimental.pallas import tpu_sc as plsc

NC, NS, LANES = 2, 16, 16
NW = NC * NS
CHUNK = 4
PLANE0 = 128
SUMU = 8

BLKV = 32768
QV = BLKV // 4
SHQ = QV.bit_length() - 1
MASK_HI = -65536


def _remap(v):
    u = v & (BLKV - 1)
    return (v - u) + ((u & (QV - 1)) << 2) + (u >> SHQ)


def _pad_body(x_ref, o_ref):
    xb = _remap(x_ref[...])
    blk = xb.shape[0]
    o_ref[0, :, :] = xb[:, :PLANE0]
    o_ref[1, :, :] = jnp.concatenate(
        [xb[:, PLANE0:], jnp.zeros((blk, 2 * PLANE0 - xb.shape[1]),
                                   jnp.int32)], axis=1)


@functools.lru_cache(maxsize=None)
def _make_padder(B, L, blk):
    return pl.pallas_call(
        _pad_body,
        grid=(B // blk,),
        in_specs=[pl.BlockSpec((blk, L), lambda i: (i, 0))],
        out_specs=pl.BlockSpec((2, blk, PLANE0), lambda i: (0, i, 0)),
        out_shape=jax.ShapeDtypeStruct((2, B, PLANE0), jnp.int32),
    )


def _rn_bf16_hi(bits):
    return (bits + 0x7FFF + ((bits >> 16) & 1)) & MASK_HI


def _fmt_body(t_ref, o_ref):
    tp = t_ref[...].T
    ah = lax.bitcast_convert_type(tp[:, :32], jnp.int32)
    al = lax.bitcast_convert_type(tp[:, 32:], jnp.int32)
    packed = _rn_bf16_hi(ah) | ((_rn_bf16_hi(al) >> 16) & 0xFFFF)
    o_ref[...] = jnp.concatenate(
        [packed[q * QV:(q + 1) * QV] for q in range(4)], axis=1)


@functools.lru_cache(maxsize=None)
def _make_fmt(V, D):
    n_blk = -(-V // BLKV)
    return pl.pallas_call(
        _fmt_body,
        grid=(n_blk,),
        in_specs=[pl.BlockSpec((D, BLKV), lambda i: (0, i))],
        out_specs=pl.BlockSpec((QV, 2 * D), lambda i: (i, 0)),
        out_shape=jax.ShapeDtypeStruct((n_blk * QV, 2 * D), jnp.int32),
    )


@functools.lru_cache(maxsize=None)
def _make_pool(B, L, D):
    b_per_w = B // NW
    n_chunks = b_per_w // CHUNK
    n_dv = D // LANES
    n_pv = D // (2 * LANES)
    l1 = L - PLANE0
    n_cv1 = -(-l1 // LANES)
    mesh = plsc.VectorSubcoreMesh(core_axis_name="c", subcore_axis_name="s")

    @functools.partial(
        pl.kernel,
        out_type=(
            jax.ShapeDtypeStruct((B, D), jnp.float32),
            jax.ShapeDtypeStruct((B, LANES), jnp.float32),
        ),
        mesh=mesh,
        compiler_params=pltpu.CompilerParams(use_tc_tiling_on_sc=False,
                                             needs_layout_passes=False),
        scratch_types=[
            pltpu.VMEM((CHUNK, PLANE0), jnp.int32),
            pltpu.VMEM((CHUNK, PLANE0), jnp.int32),
            pltpu.VMEM((CHUNK, PLANE0), jnp.int32),
            pltpu.VMEM((CHUNK, PLANE0), jnp.int32),
            pltpu.VMEM((CHUNK * L, D // 2), jnp.int32),
            pltpu.VMEM((CHUNK * L, D // 2), jnp.int32),
            pltpu.VMEM((CHUNK, D), jnp.float32),
            pltpu.VMEM((CHUNK, LANES), jnp.float32),
            pltpu.SemaphoreType.DMA,
            pltpu.SemaphoreType.DMA,
        ],
    )
    def pool(x2_hbm, table_hbm, sum_hbm, cnt_hbm,
             idxA0, idxB0, idxA1, idxB1, rows0, rows1, sum_v, cnt_v,
             semA, semB):
        wid = lax.axis_index("s") * NC + lax.axis_index("c")
        w_base = wid * b_per_w

        def fire(c, idxA, idxB, rows, sem):
            row0 = w_base + c * CHUNK
            pltpu.sync_copy(x2_hbm.at[0, pl.ds(row0, CHUNK)], idxA)
            pltpu.sync_copy(x2_hbm.at[1, pl.ds(row0, CHUNK)], idxB)
            for j in range(CHUNK):
                pltpu.async_copy(
                    table_hbm.at[idxA.at[j]],
                    rows.at[pl.ds(j * L, PLANE0)], sem)
                pltpu.async_copy(
                    table_hbm.at[idxB.at[j, pl.ds(0, l1)]],
                    rows.at[pl.ds(j * L + PLANE0, l1)], sem)

        def drain(idxA, idxB, rows, sem):
            for j in range(CHUNK):
                pltpu.make_async_copy(
                    table_hbm.at[idxA.at[j]],
                    rows.at[pl.ds(j * L, PLANE0)], sem).wait()
                pltpu.make_async_copy(
                    table_hbm.at[idxB.at[j, pl.ds(0, l1)]],
                    rows.at[pl.ds(j * L + PLANE0, l1)], sem).wait()

        def compute(c, idxA, idxB, rows):
            row0 = w_base + c * CHUNK
            for j in range(CHUNK):
                base = j * L

                def sum_body(t, acc):
                    l = base + t * SUMU
                    acc = list(acc)
                    for k in range(SUMU):
                        for q in range(n_pv):
                            v = rows[l + k, pl.ds(q * LANES, LANES)]
                            acc[q] = acc[q] + plsc.bitcast(
                                v & MASK_HI, jnp.float32)
                            acc[n_pv + q] = acc[n_pv + q] + plsc.bitcast(
                                lax.shift_left(v, 16), jnp.float32)
                    return tuple(acc)

                acc = lax.fori_loop(
                    0, L // SUMU, sum_body,
                    tuple(jnp.zeros((LANES,), jnp.float32)
                          for _ in range(n_dv)),
                )
                for d in range(n_dv):
                    sum_v[j, pl.ds(d * LANES, LANES)] = acc[d]

                cv = jnp.zeros((LANES,), jnp.float32)
                for k in range(PLANE0 // LANES):
                    v = idxA[j, pl.ds(k * LANES, LANES)]
                    cv = cv + jnp.where(v != 0, 1.0, 0.0)
                for k in range(n_cv1):
                    v = idxB[j, pl.ds(k * LANES, LANES)]
                    cv = cv + jnp.where(v != 0, 1.0, 0.0)
                cnt_v[j, :] = cv

            pltpu.sync_copy(sum_v, sum_hbm.at[pl.ds(row0, CHUNK)])
            pltpu.sync_copy(cnt_v, cnt_hbm.at[pl.ds(row0, CHUNK)])

        fire(0, idxA0, idxB0, rows0, semA)

        def body(g, carry):
            c0 = 2 * g
            fire(c0 + 1, idxA1, idxB1, rows1, semB)
            drain(idxA0, idxB0, rows0, semA)
            compute(c0, idxA0, idxB0, rows0)

            @pl.when(g + 1 < n_chunks // 2)
            def _():
                fire(c0 + 2, idxA0, idxB0, rows0, semA)

            drain(idxA1, idxB1, rows1, semB)
            compute(c0 + 1, idxA1, idxB1, rows1)
            return carry

        lax.fori_loop(0, n_chunks // 2, body, 0)

    return pool


def _head_body(s_ref, c_ref, w1_ref, b1_ref, w2_ref, b2_ref, o_ref):
    cnt = jnp.maximum(jnp.sum(c_ref[...], axis=1, keepdims=True), 1.0)
    p = s_ref[...] / cnt
    h = lax.dot_general(p, w1_ref[...], (((1,), (1,)), ((), ())),
                        preferred_element_type=jnp.float32) + b1_ref[...]
    h = 0.5 * h * (1.0 + lax.erf(h * (2.0 ** -0.5)))
    o_ref[...] = lax.dot_general(h, w2_ref[...], (((1,), (1,)), ((), ())),
                                 preferred_element_type=jnp.float32) + b2_ref[...]


@functools.lru_cache(maxsize=None)
def _make_head(B, D, C, blk):
    return pl.pallas_call(
        _head_body,
        grid=(B // blk,),
        in_specs=[
            pl.BlockSpec((blk, D), lambda i: (i, 0)),
            pl.BlockSpec((blk, LANES), lambda i: (i, 0)),
            pl.BlockSpec((D, D), lambda i: (0, 0)),
            pl.BlockSpec((1, D), lambda i: (0, 0)),
            pl.BlockSpec((C, D), lambda i: (0, 0)),
            pl.BlockSpec((1, C), lambda i: (0, 0)),
        ],
        out_specs=pl.BlockSpec((blk, C), lambda i: (i, 0)),
        out_shape=jax.ShapeDtypeStruct((B, C), jnp.float32),
    )


def kernel(x, table, W1, b1, W2, b2):
    B, L = x.shape
    V, D = table.shape
    C = W2.shape[0]
    n_blk = -(-V // BLKV)
    x2 = _make_padder(B, L, 1024)(x.astype(jnp.int32))
    tl = _make_fmt(V, D)(table.T).reshape(n_blk * BLKV, D // 2)
    sums, cnts = _make_pool(B, L, D)(x2, tl)
    return _make_head(B, D, C, 512)(
        sums, cnts, W1, b1.reshape(1, D), W2, b2.reshape(1, C))


# speedup vs baseline: 1.5278x; 1.5278x over previous
import functools

import jax
import jax.numpy as jnp
from jax import lax
from jax.experimental import pallas as pl
from jax.experimental.pallas import tpu as pltpu
from jax.experimental.pallas import tpu_sc as plsc

NC, NS, LANES = 2, 16, 16
NW = NC * NS
CHUNK = 4
PLANE0 = 128
SUMU = 8


BLKV = 32768
HALFV = BLKV // 2


def _remap(v):
    u = v & (BLKV - 1)
    return (v - u) + ((u & (HALFV - 1)) << 1) + jnp.where(u >= HALFV, 1, 0)


def _pad_body(x_ref, o_ref, c_ref):
    raw = x_ref[...]
    xb = _remap(raw)
    blk = xb.shape[0]
    o_ref[0, :, :] = xb[:, :PLANE0]
    o_ref[1, :, :] = jnp.concatenate(
        [xb[:, PLANE0:], jnp.zeros((blk, 2 * PLANE0 - xb.shape[1]),
                                   jnp.int32)], axis=1)
    c_ref[...] = jnp.sum(jnp.where(raw != 0, 1.0, 0.0), axis=1,
                         keepdims=True)


@functools.lru_cache(maxsize=None)
def _make_padder(B, L, blk):
    return pl.pallas_call(
        _pad_body,
        grid=(B // blk,),
        in_specs=[pl.BlockSpec((blk, L), lambda i: (i, 0))],
        out_specs=[
            pl.BlockSpec((2, blk, PLANE0), lambda i: (0, i, 0)),
            pl.BlockSpec((blk, 1), lambda i: (i, 0)),
        ],
        out_shape=[
            jax.ShapeDtypeStruct((2, B, PLANE0), jnp.int32),
            jax.ShapeDtypeStruct((B, 1), jnp.float32),
        ],
    )


def _fmt_body(t_ref, o_ref):
    tp = t_ref[...].T
    o_ref[...] = jnp.concatenate([tp[:HALFV], tp[HALFV:]], axis=1)


@functools.lru_cache(maxsize=None)
def _make_fmt(V, D):
    n_blk = -(-V // BLKV)
    return pl.pallas_call(
        _fmt_body,
        grid=(n_blk,),
        in_specs=[pl.BlockSpec((D, BLKV), lambda i: (0, i))],
        out_specs=pl.BlockSpec((HALFV, 2 * D), lambda i: (i, 0)),
        out_shape=jax.ShapeDtypeStruct((n_blk * HALFV, 2 * D), jnp.float32),
    )


@functools.lru_cache(maxsize=None)
def _make_pool(B, L, D):
    b_per_w = B // NW
    n_chunks = b_per_w // CHUNK
    n_dv = D // LANES
    l1 = L - PLANE0
    n_cv1 = -(-l1 // LANES)
    mesh = plsc.VectorSubcoreMesh(core_axis_name="c", subcore_axis_name="s")

    @functools.partial(
        pl.kernel,
        out_type=jax.ShapeDtypeStruct((B, D), jnp.float32),
        mesh=mesh,
        compiler_params=pltpu.CompilerParams(use_tc_tiling_on_sc=False,
                                             needs_layout_passes=False),
        scratch_types=[
            pltpu.VMEM((CHUNK, PLANE0), jnp.int32),
            pltpu.VMEM((CHUNK, PLANE0), jnp.int32),
            pltpu.VMEM((CHUNK, PLANE0), jnp.int32),
            pltpu.VMEM((CHUNK, PLANE0), jnp.int32),
            pltpu.VMEM((CHUNK * L, D), jnp.float32),
            pltpu.VMEM((CHUNK * L, D), jnp.float32),
            pltpu.VMEM((CHUNK, D), jnp.float32),
            pltpu.SemaphoreType.DMA,
            pltpu.SemaphoreType.DMA,
            pltpu.SemaphoreType.DMA,
            pltpu.SemaphoreType.DMA,
        ],
    )
    def pool(x2_hbm, table_hbm, sum_hbm,
             idxA0, idxB0, idxA1, idxB1, rows0, rows1, sum_v,
             semA, semB, semI0, semI1):
        wid = lax.axis_index("s") * NC + lax.axis_index("c")
        w_base = wid * b_per_w

        def fire_idx(c, idxA, idxB, semi):
            row0 = w_base + c * CHUNK
            pltpu.async_copy(x2_hbm.at[0, pl.ds(row0, CHUNK)], idxA, semi)
            pltpu.async_copy(x2_hbm.at[1, pl.ds(row0, CHUNK)], idxB, semi)

        def wait_idx(c, idxA, idxB, semi):
            row0 = w_base + c * CHUNK
            pltpu.make_async_copy(
                x2_hbm.at[0, pl.ds(row0, CHUNK)], idxA, semi).wait()
            pltpu.make_async_copy(
                x2_hbm.at[1, pl.ds(row0, CHUNK)], idxB, semi).wait()

        def fire_gather(idxA, idxB, rows, sem):
            for j in range(CHUNK):
                pltpu.async_copy(
                    table_hbm.at[idxA.at[j]],
                    rows.at[pl.ds(j * L, PLANE0)], sem)
                pltpu.async_copy(
                    table_hbm.at[idxB.at[j, pl.ds(0, l1)]],
                    rows.at[pl.ds(j * L + PLANE0, l1)], sem)

        def drain(idxA, idxB, rows, sem):
            for j in range(CHUNK):
                pltpu.make_async_copy(
                    table_hbm.at[idxA.at[j]],
                    rows.at[pl.ds(j * L, PLANE0)], sem).wait()
                pltpu.make_async_copy(
                    table_hbm.at[idxB.at[j, pl.ds(0, l1)]],
                    rows.at[pl.ds(j * L + PLANE0, l1)], sem).wait()

        def compute(c, rows):
            row0 = w_base + c * CHUNK
            for j in range(CHUNK):
                base = j * L

                def sum_body(t, acc):
                    l = base + t * SUMU
                    for k in range(SUMU):
                        acc = tuple(
                            acc[d] + rows[l + k, pl.ds(d * LANES, LANES)]
                            for d in range(n_dv)
                        )
                    return acc

                acc = lax.fori_loop(
                    0, L // SUMU, sum_body,
                    tuple(jnp.zeros((LANES,), jnp.float32)
                          for _ in range(n_dv)),
                )
                for d in range(n_dv):
                    sum_v[j, pl.ds(d * LANES, LANES)] = acc[d]

            pltpu.sync_copy(sum_v, sum_hbm.at[pl.ds(row0, CHUNK)])

        n2 = n_chunks // 2
        fire_idx(0, idxA0, idxB0, semI0)
        wait_idx(0, idxA0, idxB0, semI0)
        fire_gather(idxA0, idxB0, rows0, semA)
        fire_idx(1, idxA1, idxB1, semI1)

        def body(g, carry):
            c0 = 2 * g
            wait_idx(c0 + 1, idxA1, idxB1, semI1)
            fire_gather(idxA1, idxB1, rows1, semB)
            drain(idxA0, idxB0, rows0, semA)

            @pl.when(g + 1 < n2)
            def _():
                fire_idx(c0 + 2, idxA0, idxB0, semI0)

            compute(c0, rows0)

            @pl.when(g + 1 < n2)
            def _():
                wait_idx(c0 + 2, idxA0, idxB0, semI0)
                fire_gather(idxA0, idxB0, rows0, semA)

            drain(idxA1, idxB1, rows1, semB)

            @pl.when(g + 1 < n2)
            def _():
                fire_idx(c0 + 3, idxA1, idxB1, semI1)

            compute(c0 + 1, rows1)
            return carry

        lax.fori_loop(0, n2, body, 0)

    return pool


def _head_body(s_ref, c_ref, w1_ref, b1_ref, w2_ref, b2_ref, o_ref):
    cnt = jnp.maximum(c_ref[...], 1.0)
    p = s_ref[...] / cnt
    h = lax.dot_general(p, w1_ref[...], (((1,), (1,)), ((), ())),
                        preferred_element_type=jnp.float32) + b1_ref[...]
    h = 0.5 * h * (1.0 + lax.erf(h * (2.0 ** -0.5)))
    o_ref[...] = lax.dot_general(h, w2_ref[...], (((1,), (1,)), ((), ())),
                                 preferred_element_type=jnp.float32) + b2_ref[...]


@functools.lru_cache(maxsize=None)
def _make_head(B, D, C, blk):
    return pl.pallas_call(
        _head_body,
        grid=(B // blk,),
        in_specs=[
            pl.BlockSpec((blk, D), lambda i: (i, 0)),
            pl.BlockSpec((blk, 1), lambda i: (i, 0)),
            pl.BlockSpec((D, D), lambda i: (0, 0)),
            pl.BlockSpec((1, D), lambda i: (0, 0)),
            pl.BlockSpec((C, D), lambda i: (0, 0)),
            pl.BlockSpec((1, C), lambda i: (0, 0)),
        ],
        out_specs=pl.BlockSpec((blk, C), lambda i: (i, 0)),
        out_shape=jax.ShapeDtypeStruct((B, C), jnp.float32),
    )


def kernel(x, table, W1, b1, W2, b2):
    B, L = x.shape
    V, D = table.shape
    C = W2.shape[0]
    x2, cnts = _make_padder(B, L, 1024)(x.astype(jnp.int32))
    vpad = -(-V // BLKV) * BLKV
    tl = _make_fmt(V, D)(table.T).reshape(vpad, D)
    sums = _make_pool(B, L, D)(x2, tl)
    return _make_head(B, D, C, 512)(
        sums, cnts, W1, b1.reshape(1, D), W2, b2.reshape(1, C))
